# baseline (speedup 1.0000x reference)
import functools

import jax
import jax.numpy as jnp
from jax import lax
from jax.experimental import pallas as pl
from jax.experimental.pallas import tpu as pltpu
from jax.experimental.pallas import tpu_sc as plsc

_R, _C = 2048, 4096
_NC, _NS = 2, 16
_NW = _NC * _NS
_ROWS_PER_W = _R // _NW
_CHUNK = 4
_NCHUNK = _ROWS_PER_W // _CHUNK
_NBUF = 2

_mesh = plsc.VectorSubcoreMesh(core_axis_name="c", subcore_axis_name="s")


@functools.partial(
    pl.kernel,
    mesh=_mesh,
    out_type=jax.ShapeDtypeStruct((_R, _C), jnp.float32),
    scratch_types=(
        [pltpu.VMEM_SHARED((_NS, _NBUF, _CHUNK, _C), jnp.float32)]
        + [pltpu.SemaphoreType.DMA for _ in range(2 * _NBUF)]
    ),
)
def _sc_copy(in_hbm, out_hbm, shared, *sems):
    isems = sems[:_NBUF]
    osems = sems[_NBUF:]
    sid = lax.axis_index("s")
    wid = sid * _NC + lax.axis_index("c")
    base = wid * _ROWS_PER_W

    def row_slice(c):
        return pl.ds(base + c * _CHUNK, _CHUNK)

    ins = [None] * _NCHUNK
    outs = [None] * _NCHUNK
    for c in range(min(_NBUF, _NCHUNK)):
        ins[c] = pltpu.async_copy(in_hbm.at[row_slice(c)], shared.at[sid, c], isems[c])
    for c in range(_NCHUNK):
        b = c % _NBUF
        if c >= _NBUF:
            outs[c - _NBUF].wait()
            ins[c] = pltpu.async_copy(in_hbm.at[row_slice(c)], shared.at[sid, b], isems[b])
        ins[c].wait()
        outs[c] = pltpu.async_copy(shared.at[sid, b], out_hbm.at[row_slice(c)], osems[b])
    for c in range(max(0, _NCHUNK - _NBUF), _NCHUNK):
        outs[c].wait()


def kernel(free_values):
    return _sc_copy(free_values.reshape(_R, _C))


# speedup vs baseline: 1.3129x; 1.3129x over previous
import jax
import jax.numpy as jnp
from jax.experimental import pallas as pl
from jax.experimental.pallas import tpu as pltpu

_R, _C = 2048, 4096
_BLK = 128


def _copy_body(in_ref, out_ref):
    out_ref[...] = in_ref[...]


def kernel(free_values):
    x = free_values.reshape(_R, _C)
    return pl.pallas_call(
        _copy_body,
        grid=(_R // _BLK,),
        in_specs=[pl.BlockSpec((_BLK, _C), lambda i: (i, 0))],
        out_specs=pl.BlockSpec((_BLK, _C), lambda i: (i, 0)),
        out_shape=jax.ShapeDtypeStruct((_R, _C), jnp.float32),
    )(x)
